# Initial kernel scaffold; baseline (speedup 1.0000x reference)
import jax
import jax.numpy as jnp
from jax.experimental import pallas as pl


def kernel(x_prime, u):
    raise NotImplementedError("write your pallas kernel here")


# speedup vs baseline: 3.1015x; 3.1015x over previous
import functools

import jax
import jax.numpy as jnp
from jax import lax
from jax.experimental import pallas as pl
from jax.experimental.pallas import tpu as pltpu
from jax.experimental.pallas import tpu_sc as plsc

NUM_GRID = 64
C = 1536
N = 65536
L = 16
NC, NS = 2, 16
NW = NC * NS
PTS_PER_W = N // NW
CHUNK = 16
NCHUNK = PTS_PER_W // CHUNK
CV = C // L

_mesh = plsc.VectorSubcoreMesh(core_axis_name="c", subcore_axis_name="s")


@functools.partial(
    pl.kernel,
    out_type=jax.ShapeDtypeStruct((N, C), jnp.float32),
    mesh=_mesh,
    scratch_types=[
        pltpu.VMEM((PTS_PER_W,), jnp.float32),
        pltpu.VMEM((PTS_PER_W,), jnp.float32),
        pltpu.VMEM((PTS_PER_W,), jnp.int32),
        pltpu.VMEM((PTS_PER_W,), jnp.int32),
        pltpu.VMEM((PTS_PER_W,), jnp.int32),
        pltpu.VMEM((PTS_PER_W,), jnp.int32),
        pltpu.VMEM((PTS_PER_W,), jnp.float32),
        pltpu.VMEM((PTS_PER_W,), jnp.float32),
        pltpu.VMEM((CHUNK, C), jnp.float32),
        pltpu.VMEM((CHUNK, C), jnp.float32),
        pltpu.VMEM((CHUNK, C), jnp.float32),
        pltpu.VMEM((CHUNK, C), jnp.float32),
        pltpu.SemaphoreType.DMA,
    ],
)
def _decode(x_hbm, y_hbm, tab_hbm, out_hbm,
            xv, yv, i00, i01, i10, i11, wxv, wyv,
            b00, b01, b10, b11, sem):
    wid = lax.axis_index("s") * NC + lax.axis_index("c")
    base = wid * PTS_PER_W
    pltpu.sync_copy(x_hbm.at[pl.ds(base, PTS_PER_W)], xv)
    pltpu.sync_copy(y_hbm.at[pl.ds(base, PTS_PER_W)], yv)

    def idx_body(i, carry):
        s = pl.ds(i * L, L)
        x = xv[s]
        y = yv[s]
        xi = jnp.clip(x.astype(jnp.int32), 0, NUM_GRID - 2)
        yi = jnp.clip(y.astype(jnp.int32), 0, NUM_GRID - 2)
        wxv[s] = x - xi.astype(jnp.float32)
        wyv[s] = y - yi.astype(jnp.float32)
        cell = xi * NUM_GRID + yi
        i00[s] = cell
        i01[s] = cell + 1
        i10[s] = cell + NUM_GRID
        i11[s] = cell + NUM_GRID + 1
        return carry

    lax.fori_loop(0, PTS_PER_W // L, idx_body, 0)

    def chunk_body(g, carry):
        o = g * CHUNK
        cp0 = pltpu.async_copy(tab_hbm.at[i00.at[pl.ds(o, CHUNK)]], b00, sem)
        cp1 = pltpu.async_copy(tab_hbm.at[i01.at[pl.ds(o, CHUNK)]], b01, sem)
        cp2 = pltpu.async_copy(tab_hbm.at[i10.at[pl.ds(o, CHUNK)]], b10, sem)
        cp3 = pltpu.async_copy(tab_hbm.at[i11.at[pl.ds(o, CHUNK)]], b11, sem)
        wxc = wxv[pl.ds(o, L)]
        wyc = wyv[pl.ds(o, L)]
        w00c = (1.0 - wxc) * (1.0 - wyc)
        w01c = (1.0 - wxc) * wyc
        w10c = wxc * (1.0 - wyc)
        w11c = wxc * wyc
        cp0.wait()
        cp1.wait()
        cp2.wait()
        cp3.wait()

        for j in range(CHUNK):
            w00 = w00c[j]
            w01 = w01c[j]
            w10 = w10c[j]
            w11 = w11c[j]

            def col_body(c, carry2, j=j, w00=w00, w01=w01, w10=w10, w11=w11):
                cs = pl.ds(c * L, L)
                b00[j, cs] = (b00[j, cs] * w00 + b01[j, cs] * w01
                              + b10[j, cs] * w10 + b11[j, cs] * w11)
                return carry2

            lax.fori_loop(0, CV, col_body, 0)

        pltpu.sync_copy(b00, out_hbm.at[pl.ds(base + o, CHUNK)])
        return carry

    lax.fori_loop(0, NCHUNK, chunk_body, 0)


def kernel(x_prime, u):
    tab = u.transpose(1, 2, 0).reshape(NUM_GRID * NUM_GRID, C)
    return _decode(x_prime[:, 0], x_prime[:, 1], tab)


# speedup vs baseline: 5.7230x; 1.8452x over previous
import functools

import jax
import jax.numpy as jnp
from jax import lax
from jax.experimental import pallas as pl
from jax.experimental.pallas import tpu as pltpu
from jax.experimental.pallas import tpu_sc as plsc

NUM_GRID = 64
C = 1536
N = 65536
L = 16
NC, NS = 2, 16
NW = NC * NS
PTS_PER_W = N // NW
CHUNK = 8
SETS = 2
NCHUNK = PTS_PER_W // CHUNK
NPAIR = NCHUNK // SETS
CV = C // L

_mesh = plsc.VectorSubcoreMesh(core_axis_name="c", subcore_axis_name="s")


@functools.partial(
    pl.kernel,
    out_type=jax.ShapeDtypeStruct((N, C), jnp.float32),
    mesh=_mesh,
    scratch_types=[
        pltpu.VMEM((PTS_PER_W,), jnp.float32),
        pltpu.VMEM((PTS_PER_W,), jnp.float32),
        pltpu.VMEM((PTS_PER_W,), jnp.int32),
        pltpu.VMEM((PTS_PER_W,), jnp.int32),
        pltpu.VMEM((PTS_PER_W,), jnp.int32),
        pltpu.VMEM((PTS_PER_W,), jnp.int32),
        pltpu.VMEM((PTS_PER_W + L,), jnp.float32),
        pltpu.VMEM((PTS_PER_W + L,), jnp.float32),
        pltpu.VMEM((CHUNK, C), jnp.float32),
        pltpu.VMEM((CHUNK, C), jnp.float32),
        pltpu.VMEM((CHUNK, C), jnp.float32),
        pltpu.VMEM((CHUNK, C), jnp.float32),
        pltpu.VMEM((CHUNK, C), jnp.float32),
        pltpu.VMEM((CHUNK, C), jnp.float32),
        pltpu.VMEM((CHUNK, C), jnp.float32),
        pltpu.VMEM((CHUNK, C), jnp.float32),
        pltpu.SemaphoreType.DMA,
        pltpu.SemaphoreType.DMA,
        pltpu.SemaphoreType.DMA,
    ],
)
def _decode(x_hbm, y_hbm, tab_hbm, out_hbm,
            xv, yv, i00, i01, i10, i11, wxv, wyv,
            a0, a1, a2, a3, b0, b1, b2, b3,
            gsa, gsb, osem):
    wid = lax.axis_index("s") * NC + lax.axis_index("c")
    base = wid * PTS_PER_W
    pltpu.sync_copy(x_hbm.at[pl.ds(base, PTS_PER_W)], xv)
    pltpu.sync_copy(y_hbm.at[pl.ds(base, PTS_PER_W)], yv)

    def idx_body(i, carry):
        s = pl.ds(i * L, L)
        x = xv[s]
        y = yv[s]
        xi = jnp.clip(x.astype(jnp.int32), 0, NUM_GRID - 2)
        yi = jnp.clip(y.astype(jnp.int32), 0, NUM_GRID - 2)
        wxv[s] = x - xi.astype(jnp.float32)
        wyv[s] = y - yi.astype(jnp.float32)
        cell = xi * NUM_GRID + yi
        i00[s] = cell
        i01[s] = cell + 1
        i10[s] = cell + NUM_GRID
        i11[s] = cell + NUM_GRID + 1
        return carry

    lax.fori_loop(0, PTS_PER_W // L, idx_body, 0)

    _IDX = (i00, i01, i10, i11)

    def issue_gathers(o, bufs, sem):
        for idx, buf in zip(_IDX, bufs):
            pltpu.async_copy(tab_hbm.at[idx.at[pl.ds(o, CHUNK)]], buf, sem)

    def drain(sem, buf):
        pltpu.make_async_copy(tab_hbm.at[pl.ds(0, CHUNK)], buf, sem).wait()

    def blend(o, bufs):
        wxc = wxv[pl.ds(o, L)]
        wyc = wyv[pl.ds(o, L)]
        w00c = (1.0 - wxc) * (1.0 - wyc)
        w01c = (1.0 - wxc) * wyc
        w10c = wxc * (1.0 - wyc)
        w11c = wxc * wyc
        c0, c1, c2, c3 = bufs
        for j in range(CHUNK):
            w00 = w00c[j]
            w01 = w01c[j]
            w10 = w10c[j]
            w11 = w11c[j]

            def col_body(c, carry, j=j, w00=w00, w01=w01, w10=w10, w11=w11):
                cs = pl.ds(c * L, L)
                c0[j, cs] = (c0[j, cs] * w00 + c1[j, cs] * w01
                             + c2[j, cs] * w10 + c3[j, cs] * w11)
                return carry

            lax.fori_loop(0, CV, col_body, 0, unroll=4)

    sets = ((0, (a0, a1, a2, a3), gsa), (1, (b0, b1, b2, b3), gsb))

    for par, bufs, gsem in sets:
        issue_gathers(par * CHUNK, bufs, gsem)

    def pair_body(t, carry):
        for par, bufs, gsem in sets:
            g = t * SETS + par
            o = g * CHUNK
            for buf in bufs:
                drain(gsem, buf)
            blend(o, bufs)
            pltpu.async_copy(bufs[0], out_hbm.at[pl.ds(base + o, CHUNK)], osem)
            o2 = o + SETS * CHUNK
            for idx, buf in zip(_IDX[1:], bufs[1:]):
                pltpu.async_copy(tab_hbm.at[idx.at[pl.ds(o2, CHUNK)]], buf, gsem)
            drain(osem, bufs[0])
            pltpu.async_copy(tab_hbm.at[i00.at[pl.ds(o2, CHUNK)]], bufs[0], gsem)
        return carry

    lax.fori_loop(0, NPAIR - 1, pair_body, 0)

    for par, bufs, gsem in sets:
        o = (NCHUNK - SETS + par) * CHUNK
        for buf in bufs:
            drain(gsem, buf)
        blend(o, bufs)
        pltpu.async_copy(bufs[0], out_hbm.at[pl.ds(base + o, CHUNK)], osem)
    drain(osem, a0)
    drain(osem, b0)


def kernel(x_prime, u):
    tab = u.transpose(1, 2, 0).reshape(NUM_GRID * NUM_GRID, C)
    return _decode(x_prime[:, 0], x_prime[:, 1], tab)


# speedup vs baseline: 10.9855x; 1.9195x over previous
import functools

import jax
import jax.numpy as jnp
from jax import lax
from jax.experimental import pallas as pl
from jax.experimental.pallas import tpu as pltpu
from jax.experimental.pallas import tpu_sc as plsc

NUM_GRID = 64
C = 1536
HALF = C // 2
N = 65536
L = 16
NC, NS = 2, 16
NW = NC * NS
PTS_PER_W = N // NW
CHUNK = 8
SETS = 2
NCHUNK = PTS_PER_W // CHUNK
NPAIR = NCHUNK // SETS
CPAIR = C // (2 * L)

_mesh = plsc.VectorSubcoreMesh(core_axis_name="c", subcore_axis_name="s")


@functools.partial(
    pl.kernel,
    out_type=jax.ShapeDtypeStruct((N, C), jnp.float32),
    mesh=_mesh,
    scratch_types=[
        pltpu.VMEM((PTS_PER_W,), jnp.float32),
        pltpu.VMEM((PTS_PER_W,), jnp.float32),
        pltpu.VMEM((PTS_PER_W,), jnp.int32),
        pltpu.VMEM((PTS_PER_W,), jnp.int32),
        pltpu.VMEM((PTS_PER_W,), jnp.int32),
        pltpu.VMEM((PTS_PER_W,), jnp.int32),
        pltpu.VMEM((PTS_PER_W + L,), jnp.float32),
        pltpu.VMEM((PTS_PER_W + L,), jnp.float32),
        pltpu.VMEM((CHUNK, HALF), jnp.int32),
        pltpu.VMEM((CHUNK, HALF), jnp.int32),
        pltpu.VMEM((CHUNK, HALF), jnp.int32),
        pltpu.VMEM((CHUNK, HALF), jnp.int32),
        pltpu.VMEM((CHUNK, HALF), jnp.int32),
        pltpu.VMEM((CHUNK, HALF), jnp.int32),
        pltpu.VMEM((CHUNK, HALF), jnp.int32),
        pltpu.VMEM((CHUNK, HALF), jnp.int32),
        pltpu.VMEM((CHUNK, C), jnp.float32),
        pltpu.VMEM((CHUNK, C), jnp.float32),
        pltpu.SemaphoreType.DMA,
        pltpu.SemaphoreType.DMA,
        pltpu.SemaphoreType.DMA,
        pltpu.SemaphoreType.DMA,
    ],
)
def _decode(x_hbm, y_hbm, tab_hbm, out_hbm,
            xv, yv, i00, i01, i10, i11, wxv, wyv,
            a0, a1, a2, a3, b0, b1, b2, b3, oa, ob,
            gsa, gsb, osa, osb):
    wid = lax.axis_index("s") * NC + lax.axis_index("c")
    base = wid * PTS_PER_W
    pltpu.sync_copy(x_hbm.at[pl.ds(base, PTS_PER_W)], xv)
    pltpu.sync_copy(y_hbm.at[pl.ds(base, PTS_PER_W)], yv)

    def idx_body(i, carry):
        s = pl.ds(i * L, L)
        x = xv[s]
        y = yv[s]
        xi = jnp.clip(x.astype(jnp.int32), 0, NUM_GRID - 2)
        yi = jnp.clip(y.astype(jnp.int32), 0, NUM_GRID - 2)
        wxv[s] = x - xi.astype(jnp.float32)
        wyv[s] = y - yi.astype(jnp.float32)
        cell = xi * NUM_GRID + yi
        i00[s] = cell
        i01[s] = cell + 1
        i10[s] = cell + NUM_GRID
        i11[s] = cell + NUM_GRID + 1
        return carry

    lax.fori_loop(0, PTS_PER_W // L, idx_body, 0, unroll=2)

    _IDX = (i00, i01, i10, i11)

    def issue_gathers(o, bufs, sem):
        for idx, buf in zip(_IDX, bufs):
            pltpu.async_copy(tab_hbm.at[idx.at[pl.ds(o, CHUNK)]], buf, sem)

    def drain(sem, buf):
        pltpu.make_async_copy(tab_hbm.at[pl.ds(0, CHUNK)], buf, sem).wait()

    def drain_store(sem, buf):
        pltpu.make_async_copy(out_hbm.at[pl.ds(0, CHUNK)], buf, sem).wait()

    def blend(o, bufs, obuf):
        wxc = wxv[pl.ds(o, L)]
        wyc = wyv[pl.ds(o, L)]
        w00c = (1.0 - wxc) * (1.0 - wyc)
        w01c = (1.0 - wxc) * wyc
        w10c = wxc * (1.0 - wyc)
        w11c = wxc * wyc
        c0, c1, c2, c3 = bufs
        ws = [(w00c[j], w01c[j], w10c[j], w11c[j]) for j in range(CHUNK)]
        for jj in range(0, CHUNK, 2):

            @plsc.parallel_loop(0, HALF, step=L, unroll=2)
            def col_body(ci, jj=jj):
                cs = pl.ds(ci, L)
                hs = pl.ds(HALF + ci, L)

                def widen(v):
                    lo = lax.bitcast_convert_type(v << 16, jnp.float32)
                    hi = lax.bitcast_convert_type(v, jnp.float32)
                    return lo, hi

                for j in (jj, jj + 1):
                    w00, w01, w10, w11 = ws[j]
                    l0, h0 = widen(c0[j, cs])
                    l1, h1 = widen(c1[j, cs])
                    l2, h2 = widen(c2[j, cs])
                    l3, h3 = widen(c3[j, cs])
                    obuf[j, cs] = l0 * w00 + l1 * w01 + l2 * w10 + l3 * w11
                    obuf[j, hs] = h0 * w00 + h1 * w01 + h2 * w10 + h3 * w11

    sets = ((0, (a0, a1, a2, a3), oa, gsa, osa),
            (1, (b0, b1, b2, b3), ob, gsb, osb))

    for par, bufs, obuf, gsem, osem in sets:
        issue_gathers(par * CHUNK, bufs, gsem)

    def body(t, carry, drain_prev, prefetch):
        for par, bufs, obuf, gsem, osem in sets:
            g = t * SETS + par
            o = g * CHUNK
            for buf in bufs:
                drain(gsem, buf)
            if drain_prev:
                drain_store(osem, obuf)
            blend(o, bufs, obuf)
            pltpu.async_copy(obuf, out_hbm.at[pl.ds(base + o, CHUNK)], osem)
            if prefetch:
                issue_gathers(o + SETS * CHUNK, bufs, gsem)
        return carry

    body(0, 0, drain_prev=False, prefetch=True)
    lax.fori_loop(1, NPAIR - 1,
                  functools.partial(body, drain_prev=True, prefetch=True), 0)
    body(NPAIR - 1, 0, drain_prev=True, prefetch=False)
    drain_store(osa, oa)
    drain_store(osb, ob)


def kernel(x_prime, u):
    tab = u.transpose(1, 2, 0).reshape(NUM_GRID * NUM_GRID, 2, HALF)
    tab = tab.transpose(0, 2, 1).astype(jnp.bfloat16)
    tab = lax.bitcast_convert_type(tab, jnp.int32)
    return _decode(x_prime[:, 0], x_prime[:, 1], tab)
